# Initial kernel scaffold; baseline (speedup 1.0000x reference)
import jax
import jax.numpy as jnp
from jax.experimental import pallas as pl


def kernel(logits_student, logits_teacher, target):
    raise NotImplementedError("write your pallas kernel here")


# speedup vs baseline: 3.7146x; 3.7146x over previous
import jax
import jax.numpy as jnp
from jax import lax
from jax.experimental import pallas as pl
from jax.experimental.pallas import tpu as pltpu

_B = 64
_C = 100
_K = 5
_N = _B * _C * _K
_NBLK = _N // 128
_ALPHA = 0.7


def _log_softmax(x):
    m = jnp.max(x, axis=1, keepdims=True)
    s = jnp.exp(x - m)
    return (x - m) - jnp.log(jnp.sum(s, axis=1, keepdims=True))


def _scalars_cube_kernel(ls_ref, lt_ref, tgt_ref, part_ref, t_ref, s_ref):
    ls = ls_ref[...]
    lt = lt_ref[...]
    tgt = tgt_ref[...]

    cols = lax.broadcasted_iota(jnp.int32, (_B, _C), 1)
    onehot = cols == tgt
    lp1 = _log_softmax(ls)
    ce = -jnp.sum(jnp.where(onehot, lp1, 0.0), keepdims=True) / float(_B)

    zero = jnp.zeros((1, 1), jnp.float32)
    kd = zero
    tt = zero
    ss = zero
    ts = zero
    sub = zero
    for k in range(1, _K + 1):
        inv = 1.0 / float(k)
        lq = _log_softmax(ls * inv)
        lp = _log_softmax(lt * inv)
        q = jnp.exp(lq)
        p = jnp.exp(lp)
        kl = jnp.sum(p * (lp - lq), keepdims=True) / float(_B * _C)
        kd = kd + kl * (_ALPHA * k * k) + ce * (1.0 - _ALPHA)
        t_ref[(k - 1) * _B : k * _B, :] = p
        s_ref[(k - 1) * _B : k * _B, :] = q
        tt = tt + jnp.sum(p * p, keepdims=True)
        ss = ss + jnp.sum(q * q, keepdims=True)
        ts = ts + jnp.sum(p * q, keepdims=True)
        g = lax.dot_general(p, p, (((1,), (1,)), ((), ())),
                            preferred_element_type=jnp.float32) - \
            lax.dot_general(q, q, (((1,), (1,)), ((), ())),
                            preferred_element_type=jnp.float32)
        h = lax.dot_general(p, p, (((0,), (0,)), ((), ())),
                            preferred_element_type=jnp.float32) - \
            lax.dot_general(q, q, (((0,), (0,)), ((), ())),
                            preferred_element_type=jnp.float32)
        sub = sub + jnp.sum(g * g, keepdims=True) + jnp.sum(h * h, keepdims=True)

    l2 = 0.00025 * (tt * tt - 2.0 * ts * ts + ss * ss)
    part_ref[...] = kd + l2 + sub


def _l1_pairs_kernel(tcol_ref, scol_ref, trow_ref, srow_ref, out_ref):
    c = pl.program_id(0)
    j = pl.program_id(1)
    ab = 2 * j + c

    @pl.when(j == 0)
    def _():
        out_ref[...] = jnp.zeros_like(out_ref)

    tav = [jnp.broadcast_to(tcol_ref[r * 8 : (r + 1) * 8, :], (8, 128))
           for r in range(16)]
    sav = [jnp.broadcast_to(scol_ref[r * 8 : (r + 1) * 8, :], (8, 128))
           for r in range(16)]

    d0 = pl.multiple_of(ab * 128, 128)
    tbd = trow_ref[:, pl.ds(d0, 128)]
    sbd = srow_ref[:, pl.ds(d0, 128)]
    diag = jnp.zeros((8, 128), jnp.float32)
    for r in range(16):
        diag = diag + jnp.abs(tav[r] * tbd - sav[r] * sbd)

    def body(l, accs):
        o = pl.multiple_of(l * 128, 128)
        tb = trow_ref[:, pl.ds(o, 128)]
        sb = srow_ref[:, pl.ds(o, 128)]
        return tuple(
            accs[r] + jnp.abs(tav[r] * tb - sav[r] * sb) for r in range(16)
        )

    init = tuple(jnp.zeros((8, 128), jnp.float32) for _ in range(16))
    accs = lax.fori_loop(ab + 1, _NBLK, body, init)

    rest = accs[0]
    for r in range(1, 16):
        rest = rest + accs[r]
    combined = diag + 2.0 * rest
    out_ref[...] = out_ref[...] + jnp.sum(combined, keepdims=True)[None]


def _run_scalars_cube(ls, lt, tgt):
    return pl.pallas_call(
        _scalars_cube_kernel,
        out_shape=(
            jax.ShapeDtypeStruct((1, 1), jnp.float32),
            jax.ShapeDtypeStruct((_K * _B, _C), jnp.float32),
            jax.ShapeDtypeStruct((_K * _B, _C), jnp.float32),
        ),
        name="ckd_scalars_cube",
    )(ls, lt, tgt)


def _run_l1(tcol, scol, trow, srow):
    grid = (2, _NBLK // 2)
    return pl.pallas_call(
        _l1_pairs_kernel,
        grid=grid,
        in_specs=[
            pl.BlockSpec((128, 1), lambda c, j: (2 * j + c, 0)),
            pl.BlockSpec((128, 1), lambda c, j: (2 * j + c, 0)),
            pl.BlockSpec((8, _N), lambda c, j: (0, 0)),
            pl.BlockSpec((8, _N), lambda c, j: (0, 0)),
        ],
        out_specs=pl.BlockSpec((1, 1, 1), lambda c, j: (c, 0, 0)),
        out_shape=jax.ShapeDtypeStruct((2, 1, 1), jnp.float32),
        compiler_params=pltpu.CompilerParams(
            dimension_semantics=("parallel", "arbitrary"),
        ),
        name="ckd_l1_pairs",
    )(tcol, scol, trow, srow)


def kernel(logits_student, logits_teacher, target):
    ls = logits_student.astype(jnp.float32)
    lt = logits_teacher.astype(jnp.float32)
    tgt = target.astype(jnp.int32).reshape(_B, 1)

    part, t_mat, s_mat = _run_scalars_cube(ls, lt, tgt)

    tflat = t_mat.reshape(-1)
    sflat = s_mat.reshape(-1)
    tcol = tflat.reshape(_N, 1)
    scol = sflat.reshape(_N, 1)
    trow = jnp.broadcast_to(tflat[None, :], (8, _N))
    srow = jnp.broadcast_to(sflat[None, :], (8, _N))

    l1 = _run_l1(tcol, scol, trow, srow)
    return part.reshape(()) + 0.00025 * jnp.sum(l1)


# speedup vs baseline: 4.1373x; 1.1138x over previous
import jax
import jax.numpy as jnp
from jax import lax
from jax.experimental import pallas as pl
from jax.experimental.pallas import tpu as pltpu

_B = 64
_C = 100
_K = 5
_N = _B * _C * _K
_NBLK = _N // 128
_ALPHA = 0.7


def _log_softmax(x):
    m = jnp.max(x, axis=1, keepdims=True)
    s = jnp.exp(x - m)
    return (x - m) - jnp.log(jnp.sum(s, axis=1, keepdims=True))


def _scalars_cube_kernel(ls_ref, lt_ref, tgt_ref, part_ref, t_ref, s_ref):
    ls = ls_ref[...]
    lt = lt_ref[...]
    tgt = tgt_ref[...]

    cols = lax.broadcasted_iota(jnp.int32, (_B, _C), 1)
    onehot = cols == tgt
    lp1 = _log_softmax(ls)
    ce = -jnp.sum(jnp.where(onehot, lp1, 0.0), keepdims=True) / float(_B)

    zero = jnp.zeros((1, 1), jnp.float32)
    kd = zero
    tt = zero
    ss = zero
    ts = zero
    sub = zero
    for k in range(1, _K + 1):
        inv = 1.0 / float(k)
        lq = _log_softmax(ls * inv)
        lp = _log_softmax(lt * inv)
        q = jnp.exp(lq)
        p = jnp.exp(lp)
        kl = jnp.sum(p * (lp - lq), keepdims=True) / float(_B * _C)
        kd = kd + kl * (_ALPHA * k * k) + ce * (1.0 - _ALPHA)
        t_ref[(k - 1) * _B : k * _B, :] = p
        s_ref[(k - 1) * _B : k * _B, :] = q
        tt = tt + jnp.sum(p * p, keepdims=True)
        ss = ss + jnp.sum(q * q, keepdims=True)
        ts = ts + jnp.sum(p * q, keepdims=True)
        g = lax.dot_general(p, p, (((1,), (1,)), ((), ())),
                            preferred_element_type=jnp.float32) - \
            lax.dot_general(q, q, (((1,), (1,)), ((), ())),
                            preferred_element_type=jnp.float32)
        h = lax.dot_general(p, p, (((0,), (0,)), ((), ())),
                            preferred_element_type=jnp.float32) - \
            lax.dot_general(q, q, (((0,), (0,)), ((), ())),
                            preferred_element_type=jnp.float32)
        sub = sub + jnp.sum(g * g, keepdims=True) + jnp.sum(h * h, keepdims=True)

    l2 = 0.00025 * (tt * tt - 2.0 * ts * ts + ss * ss)
    part_ref[...] = kd + l2 + sub


_MA = 128
_NA = _N // _MA
_RG = _MA // 8


def _l1_pairs_kernel(tcol_ref, scol_ref, trow_ref, srow_ref, out_ref,
                     accs_ref, accr_ref):
    ab = pl.program_id(0)

    @pl.when(ab == 0)
    def _():
        accs_ref[...] = jnp.zeros_like(accs_ref)

    tav = [jnp.broadcast_to(tcol_ref[r * 8 : (r + 1) * 8, :], (8, 128))
           for r in range(_RG)]
    sav = [jnp.broadcast_to(scol_ref[r * 8 : (r + 1) * 8, :], (8, 128))
           for r in range(_RG)]

    o = pl.multiple_of(ab * 128, 128)
    tbd = trow_ref[:, pl.ds(o, 128)]
    sbd = srow_ref[:, pl.ds(o, 128)]
    for r in range(_RG):
        accr_ref[r * 8 : (r + 1) * 8, :] = 0.5 * jnp.abs(
            tav[r] * tbd - sav[r] * sbd)

    def body(l, carry):
        o = pl.multiple_of(l * 128, 128)
        tb = trow_ref[:, pl.ds(o, 128)]
        sb = srow_ref[:, pl.ds(o, 128)]
        for r in range(_RG):
            sl = slice(r * 8, (r + 1) * 8)
            accr_ref[sl, :] = accr_ref[sl, :] + jnp.abs(
                tav[r] * tb - sav[r] * sb)
        return carry

    lax.fori_loop(ab + 1, _NBLK, body, 0, unroll=False)

    folded = accr_ref[0:8, :]
    for r in range(1, _RG):
        folded = folded + accr_ref[r * 8 : (r + 1) * 8, :]
    accs_ref[...] = accs_ref[...] + 2.0 * folded

    @pl.when(ab == _NA - 1)
    def _():
        out_ref[...] = jnp.sum(accs_ref[...], keepdims=True)


def _run_scalars_cube(ls, lt, tgt):
    return pl.pallas_call(
        _scalars_cube_kernel,
        out_shape=(
            jax.ShapeDtypeStruct((1, 1), jnp.float32),
            jax.ShapeDtypeStruct((_K * _B, _C), jnp.float32),
            jax.ShapeDtypeStruct((_K * _B, _C), jnp.float32),
        ),
        name="ckd_scalars_cube",
    )(ls, lt, tgt)


def _run_l1(tcol, scol, trow, srow):
    return pl.pallas_call(
        _l1_pairs_kernel,
        grid=(_NA,),
        in_specs=[
            pl.BlockSpec((_MA, 1), lambda j: (j, 0)),
            pl.BlockSpec((_MA, 1), lambda j: (j, 0)),
            pl.BlockSpec((8, _N), lambda j: (0, 0)),
            pl.BlockSpec((8, _N), lambda j: (0, 0)),
        ],
        out_specs=pl.BlockSpec((1, 1), lambda j: (0, 0)),
        out_shape=jax.ShapeDtypeStruct((1, 1), jnp.float32),
        scratch_shapes=[pltpu.VMEM((8, 128), jnp.float32),
                        pltpu.VMEM((_MA, 128), jnp.float32)],
        compiler_params=pltpu.CompilerParams(
            dimension_semantics=("arbitrary",),
        ),
        name="ckd_l1_pairs",
    )(tcol, scol, trow, srow)


def kernel(logits_student, logits_teacher, target):
    ls = logits_student.astype(jnp.float32)
    lt = logits_teacher.astype(jnp.float32)
    tgt = target.astype(jnp.int32).reshape(_B, 1)

    part, t_mat, s_mat = _run_scalars_cube(ls, lt, tgt)

    tflat = t_mat.reshape(-1)
    sflat = s_mat.reshape(-1)
    tcol = tflat.reshape(_N, 1)
    scol = sflat.reshape(_N, 1)
    trow = jnp.broadcast_to(tflat[None, :], (8, _N))
    srow = jnp.broadcast_to(sflat[None, :], (8, _N))

    l1 = _run_l1(tcol, scol, trow, srow)
    return part.reshape(()) + 0.00025 * l1.reshape(())


# speedup vs baseline: 46.2443x; 11.1773x over previous
import jax
import jax.numpy as jnp
from jax import lax
from jax.experimental import pallas as pl
from jax.experimental.pallas import tpu as pltpu

_B = 64
_C = 100
_K = 5
_N = _B * _C * _K
_NBLK = _N // 128
_ALPHA = 0.7


def _log_softmax(x):
    m = jnp.max(x, axis=1, keepdims=True)
    s = jnp.exp(x - m)
    return (x - m) - jnp.log(jnp.sum(s, axis=1, keepdims=True))


def _scalars_cube_kernel(ls_ref, lt_ref, tgt_ref, part_ref, t_ref, s_ref,
                         x_ref):
    ls = ls_ref[...]
    lt = lt_ref[...]
    tgt = tgt_ref[...]

    cols = lax.broadcasted_iota(jnp.int32, (_B, _C), 1)
    onehot = cols == tgt
    lp1 = _log_softmax(ls)
    ce = -jnp.sum(jnp.where(onehot, lp1, 0.0), keepdims=True) / float(_B)

    zero = jnp.zeros((1, 1), jnp.float32)
    kd = zero
    tt = zero
    ss = zero
    ts = zero
    sub = zero
    for k in range(1, _K + 1):
        inv = 1.0 / float(k)
        lq = _log_softmax(ls * inv)
        lp = _log_softmax(lt * inv)
        q = jnp.exp(lq)
        p = jnp.exp(lp)
        kl = jnp.sum(p * (lp - lq), keepdims=True) / float(_B * _C)
        kd = kd + kl * (_ALPHA * k * k) + ce * (1.0 - _ALPHA)
        t_ref[(k - 1) * _B : k * _B, :] = p
        s_ref[(k - 1) * _B : k * _B, :] = q
        x_ref[(k - 1) * _B : k * _B, :] = lp - lq
        tt = tt + jnp.sum(p * p, keepdims=True)
        ss = ss + jnp.sum(q * q, keepdims=True)
        ts = ts + jnp.sum(p * q, keepdims=True)
        g = lax.dot_general(p, p, (((1,), (1,)), ((), ())),
                            preferred_element_type=jnp.float32) - \
            lax.dot_general(q, q, (((1,), (1,)), ((), ())),
                            preferred_element_type=jnp.float32)
        h = lax.dot_general(p, p, (((0,), (0,)), ((), ())),
                            preferred_element_type=jnp.float32) - \
            lax.dot_general(q, q, (((0,), (0,)), ((), ())),
                            preferred_element_type=jnp.float32)
        sub = sub + jnp.sum(g * g, keepdims=True) + jnp.sum(h * h, keepdims=True)

    l2 = 0.00025 * (tt * tt - 2.0 * ts * ts + ss * ss)
    part_ref[...] = kd + l2 + sub


_RS = 256
_NSORT = _RS * 128
_PAD_ROWS = _RS - _NBLK
_BIG = 1e30


def _partner(arr, j, lanes):
    if j < 128:
        fwd = pltpu.roll(arr, 128 - j, axis=1)
        bwd = pltpu.roll(arr, j, axis=1)
        return jnp.where((lanes & j) == 0, fwd, bwd)
    s = j // 128
    rr = arr.shape[0]
    y = arr.reshape(rr // (2 * s), 2 * s, 128)
    yswap = jnp.concatenate([y[:, s:, :], y[:, :s, :]], axis=1)
    return yswap.reshape(rr, 128)


def _substage(key, payloads, j, notasc, flat, lanes):
    pk = _partner(key, j, lanes)
    lower = (flat & j) == 0
    conda = lower if notasc is None else jnp.logical_xor(lower, notasc)
    take = (conda & (pk < key)) | (jnp.logical_not(conda) & (pk > key))
    newkey = jnp.where(take, pk, key)
    newp = [jnp.where(take, _partner(p, j, lanes), p) for p in payloads]
    return newkey, newp


def _inclusive_scan(x, lanes):
    n = x.shape[0] * 128
    d = 1
    while d < n:
        if d < 128:
            r = pltpu.roll(x, d, axis=1)
            rprev = jnp.concatenate(
                [jnp.zeros((1, 128), x.dtype), r[:-1]], axis=0)
            sh = jnp.where(lanes < d, rprev, r)
        else:
            m = d // 128
            sh = jnp.concatenate(
                [jnp.zeros((m, 128), x.dtype), x[:-m]], axis=0)
        x = x + sh
        d *= 2
    return x


def _l1_sorted_kernel(x_ref, t_ref, s_ref, out_ref):
    pad_k = jnp.full((_PAD_ROWS, 128), _BIG, jnp.float32)
    pad_z = jnp.zeros((_PAD_ROWS, 128), jnp.float32)
    key = jnp.concatenate([x_ref[...], pad_k], axis=0)
    u = jnp.concatenate([t_ref[...], pad_z], axis=0)
    v = jnp.concatenate([s_ref[...], pad_z], axis=0)

    rows = lax.broadcasted_iota(jnp.int32, (_RS, 128), 0)
    lanes = lax.broadcasted_iota(jnp.int32, (_RS, 128), 1)
    flat = rows * 128 + lanes

    size = 2
    while size <= _NSORT:
        notasc = (flat & size) != 0
        j = size // 2
        while j >= 1:
            key, (u, v) = _substage(key, [u, v], j, notasc, flat, lanes)
            j //= 2
        size *= 2

    key2 = jnp.concatenate([key, -key], axis=0)
    one = jnp.ones((_RS, 128), jnp.float32)
    f2 = jnp.concatenate([one, 1.0 - one], axis=0)
    u2 = jnp.concatenate([u, u], axis=0)
    v2 = jnp.concatenate([v, v], axis=0)

    rows2 = lax.broadcasted_iota(jnp.int32, (2 * _RS, 128), 0)
    lanes2 = lax.broadcasted_iota(jnp.int32, (2 * _RS, 128), 1)
    flat2 = rows2 * 128 + lanes2

    j = _NSORT
    while j >= 1:
        key2, (f2, u2, v2) = _substage(key2, [f2, u2, v2], j, None,
                                       flat2, lanes2)
        j //= 2

    pu = _inclusive_scan(u2 * f2, lanes2)
    pv = _inclusive_scan(v2 * f2, lanes2)
    utot = jnp.sum(u2 * f2, keepdims=True)
    vtot = jnp.sum(v2 * f2, keepdims=True)

    terms = (1.0 - f2) * (u2 * (utot - 2.0 * pu) - v2 * (vtot - 2.0 * pv))
    out_ref[...] = jnp.sum(terms, keepdims=True)


def _run_scalars_cube(ls, lt, tgt):
    return pl.pallas_call(
        _scalars_cube_kernel,
        out_shape=(
            jax.ShapeDtypeStruct((1, 1), jnp.float32),
            jax.ShapeDtypeStruct((_K * _B, _C), jnp.float32),
            jax.ShapeDtypeStruct((_K * _B, _C), jnp.float32),
            jax.ShapeDtypeStruct((_K * _B, _C), jnp.float32),
        ),
        name="ckd_scalars_cube",
    )(ls, lt, tgt)


def _run_l1(xm, tm, sm):
    return pl.pallas_call(
        _l1_sorted_kernel,
        out_shape=jax.ShapeDtypeStruct((1, 1), jnp.float32),
        name="ckd_l1_sorted",
    )(xm, tm, sm)


def kernel(logits_student, logits_teacher, target):
    ls = logits_student.astype(jnp.float32)
    lt = logits_teacher.astype(jnp.float32)
    tgt = target.astype(jnp.int32).reshape(_B, 1)

    part, t_mat, s_mat, x_mat = _run_scalars_cube(ls, lt, tgt)

    tm = t_mat.reshape(_NBLK, 128)
    sm = s_mat.reshape(_NBLK, 128)
    xm = x_mat.reshape(_NBLK, 128)

    l1 = _run_l1(xm, tm, sm)
    return part.reshape(()) + 0.00025 * l1.reshape(())


# speedup vs baseline: 48.4388x; 1.0475x over previous
import jax
import jax.numpy as jnp
from jax import lax
from jax.experimental import pallas as pl
from jax.experimental.pallas import tpu as pltpu

_B = 64
_C = 100
_K = 5
_N = _B * _C * _K
_NBLK = _N // 128
_ALPHA = 0.7


def _log_softmax(x):
    m = jnp.max(x, axis=1, keepdims=True)
    s = jnp.exp(x - m)
    return (x - m) - jnp.log(jnp.sum(s, axis=1, keepdims=True))


def _scalars_cube_kernel(ls_ref, lt_ref, tgt_ref, part_ref, cube_ref):
    ls = ls_ref[...]
    lt = lt_ref[...]
    tgt = tgt_ref[...]

    cols = lax.broadcasted_iota(jnp.int32, (_B, _C), 1)
    onehot = cols == tgt
    lp1 = _log_softmax(ls)
    ce = -jnp.sum(jnp.where(onehot, lp1, 0.0), keepdims=True) / float(_B)

    zero = jnp.zeros((1, 1), jnp.float32)
    kd = zero
    tt = zero
    ss = zero
    ts = zero
    sub = zero
    for k in range(1, _K + 1):
        inv = 1.0 / float(k)
        lq = _log_softmax(ls * inv)
        lp = _log_softmax(lt * inv)
        q = jnp.exp(lq)
        p = jnp.exp(lp)
        kl = jnp.sum(p * (lp - lq), keepdims=True) / float(_B * _C)
        kd = kd + kl * (_ALPHA * k * k) + ce * (1.0 - _ALPHA)
        base = (k - 1) * _B
        cube_ref[base : base + _B, :] = p
        cube_ref[_K * _B + base : _K * _B + base + _B, :] = q
        cube_ref[2 * _K * _B + base : 2 * _K * _B + base + _B, :] = lp - lq
        tt = tt + jnp.sum(p * p, keepdims=True)
        ss = ss + jnp.sum(q * q, keepdims=True)
        ts = ts + jnp.sum(p * q, keepdims=True)
        g = lax.dot_general(p, p, (((1,), (1,)), ((), ())),
                            preferred_element_type=jnp.float32) - \
            lax.dot_general(q, q, (((1,), (1,)), ((), ())),
                            preferred_element_type=jnp.float32)
        h = lax.dot_general(p, p, (((0,), (0,)), ((), ())),
                            preferred_element_type=jnp.float32) - \
            lax.dot_general(q, q, (((0,), (0,)), ((), ())),
                            preferred_element_type=jnp.float32)
        sub = sub + jnp.sum(g * g, keepdims=True) + jnp.sum(h * h, keepdims=True)

    l2 = 0.00025 * (tt * tt - 2.0 * ts * ts + ss * ss)
    part_ref[...] = kd + l2 + sub


_RS = 256
_NSORT = _RS * 128
_PAD_ROWS = _RS - _NBLK
_BIG = 1e30


def _partner(arr, j, lanes):
    if j < 128:
        fwd = pltpu.roll(arr, 128 - j, axis=1)
        bwd = pltpu.roll(arr, j, axis=1)
        return jnp.where((lanes & j) == 0, fwd, bwd)
    s = j // 128
    rr = arr.shape[0]
    y = arr.reshape(rr // (2 * s), 2 * s, 128)
    yswap = jnp.concatenate([y[:, s:, :], y[:, :s, :]], axis=1)
    return yswap.reshape(rr, 128)


def _substage(key, payloads, j, notasc, flat, lanes):
    pk = _partner(key, j, lanes)
    lower = (flat & j) == 0
    conda = lower if notasc is None else jnp.logical_xor(lower, notasc)
    take = (conda & (pk < key)) | (jnp.logical_not(conda) & (pk > key))
    newkey = jnp.where(take, pk, key)
    newp = [jnp.where(take, _partner(p, j, lanes), p) for p in payloads]
    return newkey, newp


def _inclusive_scan(x, lanes):
    n = x.shape[0] * 128
    d = 1
    while d < n:
        if d < 128:
            r = pltpu.roll(x, d, axis=1)
            rprev = jnp.concatenate(
                [jnp.zeros((1, 128), x.dtype), r[:-1]], axis=0)
            sh = jnp.where(lanes < d, rprev, r)
        else:
            m = d // 128
            sh = jnp.concatenate(
                [jnp.zeros((m, 128), x.dtype), x[:-m]], axis=0)
        x = x + sh
        d *= 2
    return x


def _l1_sorted_kernel(cube_ref, part_ref, out_ref):
    pad_k = jnp.full((_PAD_ROWS, 128), _BIG, jnp.float32)
    pad_z = jnp.zeros((_PAD_ROWS, 128), jnp.float32)
    key = jnp.concatenate([cube_ref[2 * _NBLK : 3 * _NBLK, :], pad_k],
                          axis=0)
    u = jnp.concatenate([cube_ref[0:_NBLK, :], pad_z], axis=0)
    v = jnp.concatenate([cube_ref[_NBLK : 2 * _NBLK, :], pad_z], axis=0)

    rows = lax.broadcasted_iota(jnp.int32, (_RS, 128), 0)
    lanes = lax.broadcasted_iota(jnp.int32, (_RS, 128), 1)
    flat = rows * 128 + lanes

    size = 2
    while size <= _NSORT:
        notasc = (flat & size) != 0
        j = size // 2
        while j >= 1:
            key, (u, v) = _substage(key, [u, v], j, notasc, flat, lanes)
            j //= 2
        size *= 2

    key2 = jnp.concatenate([key, -key], axis=0)
    one = jnp.ones((_RS, 128), jnp.float32)
    f2 = jnp.concatenate([one, 1.0 - one], axis=0)
    u2 = jnp.concatenate([u, u], axis=0)
    v2 = jnp.concatenate([v, v], axis=0)

    rows2 = lax.broadcasted_iota(jnp.int32, (2 * _RS, 128), 0)
    lanes2 = lax.broadcasted_iota(jnp.int32, (2 * _RS, 128), 1)
    flat2 = rows2 * 128 + lanes2

    j = _NSORT
    while j >= 1:
        key2, (f2, u2, v2) = _substage(key2, [f2, u2, v2], j, None,
                                       flat2, lanes2)
        j //= 2

    pu = _inclusive_scan(u2 * f2, lanes2)
    pv = _inclusive_scan(v2 * f2, lanes2)
    utot = jnp.sum(u2 * f2, keepdims=True)
    vtot = jnp.sum(v2 * f2, keepdims=True)

    terms = (1.0 - f2) * (u2 * (utot - 2.0 * pu) - v2 * (vtot - 2.0 * pv))
    out_ref[...] = part_ref[...] + 0.00025 * jnp.sum(terms, keepdims=True)


def _run_scalars_cube(ls, lt, tgt):
    return pl.pallas_call(
        _scalars_cube_kernel,
        out_shape=(
            jax.ShapeDtypeStruct((1, 1), jnp.float32),
            jax.ShapeDtypeStruct((3 * _K * _B, _C), jnp.float32),
        ),
        name="ckd_scalars_cube",
    )(ls, lt, tgt)


def _run_l1(cube, part):
    return pl.pallas_call(
        _l1_sorted_kernel,
        out_shape=jax.ShapeDtypeStruct((1, 1), jnp.float32),
        name="ckd_l1_sorted",
    )(cube, part)


def kernel(logits_student, logits_teacher, target):
    ls = logits_student.astype(jnp.float32)
    lt = logits_teacher.astype(jnp.float32)
    tgt = target.astype(jnp.int32).reshape(_B, 1)

    part, cube = _run_scalars_cube(ls, lt, tgt)
    loss = _run_l1(cube.reshape(3 * _NBLK, 128), part)
    return loss.reshape(())


# speedup vs baseline: 64.7493x; 1.3367x over previous
import jax
import jax.numpy as jnp
from jax import lax
from jax.experimental import pallas as pl
from jax.experimental.pallas import tpu as pltpu

_B = 64
_C = 100
_K = 5
_N = _B * _C * _K
_NBLK = _N // 128
_ALPHA = 0.7


def _log_softmax(x):
    m = jnp.max(x, axis=1, keepdims=True)
    s = jnp.exp(x - m)
    return (x - m) - jnp.log(jnp.sum(s, axis=1, keepdims=True))


def _scalars_cube_kernel(ls_ref, lt_ref, tgt_ref, part_ref, cube_ref):
    ls = ls_ref[...]
    lt = lt_ref[...]
    tgt = tgt_ref[...]

    cols = lax.broadcasted_iota(jnp.int32, (_B, _C), 1)
    onehot = cols == tgt
    lp1 = _log_softmax(ls)
    ce = -jnp.sum(jnp.where(onehot, lp1, 0.0), keepdims=True) / float(_B)

    zero = jnp.zeros((1, 1), jnp.float32)
    kd = zero
    tt = zero
    ss = zero
    ts = zero
    sub = zero
    for k in range(1, _K + 1):
        inv = 1.0 / float(k)
        lq = _log_softmax(ls * inv)
        lp = _log_softmax(lt * inv)
        q = jnp.exp(lq)
        p = jnp.exp(lp)
        kl = jnp.sum(p * (lp - lq), keepdims=True) / float(_B * _C)
        kd = kd + kl * (_ALPHA * k * k) + ce * (1.0 - _ALPHA)
        base = (k - 1) * _B
        cube_ref[base : base + _B, :] = p
        cube_ref[_K * _B + base : _K * _B + base + _B, :] = lp - lq
        tt = tt + jnp.sum(p * p, keepdims=True)
        ss = ss + jnp.sum(q * q, keepdims=True)
        ts = ts + jnp.sum(p * q, keepdims=True)
        g = lax.dot_general(p, p, (((1,), (1,)), ((), ())),
                            preferred_element_type=jnp.float32) - \
            lax.dot_general(q, q, (((1,), (1,)), ((), ())),
                            preferred_element_type=jnp.float32)
        h = lax.dot_general(p, p, (((0,), (0,)), ((), ())),
                            preferred_element_type=jnp.float32) - \
            lax.dot_general(q, q, (((0,), (0,)), ((), ())),
                            preferred_element_type=jnp.float32)
        sub = sub + jnp.sum(g * g, keepdims=True) + jnp.sum(h * h, keepdims=True)

    l2 = 0.00025 * (tt * tt - 2.0 * ts * ts + ss * ss)
    part_ref[...] = kd + l2 + sub


_RS = 256
_NSORT = _RS * 128
_PAD_ROWS = _RS - _NBLK
_BIG = 1e30


def _partner(arr, j, lanes):
    if j < 128:
        fwd = pltpu.roll(arr, 128 - j, axis=1)
        bwd = pltpu.roll(arr, j, axis=1)
        return jnp.where((lanes & j) == 0, fwd, bwd)
    s = j // 128
    rr = arr.shape[0]
    y = arr.reshape(rr // (2 * s), 2 * s, 128)
    yswap = jnp.concatenate([y[:, s:, :], y[:, :s, :]], axis=1)
    return yswap.reshape(rr, 128)


def _substage(key, payloads, j, notasc, flat, lanes):
    pk = _partner(key, j, lanes)
    lower = (flat & j) == 0
    conda = lower if notasc is None else jnp.logical_xor(lower, notasc)
    take = (conda & (pk < key)) | (jnp.logical_not(conda) & (pk > key))
    newkey = jnp.where(take, pk, key)
    newp = [jnp.where(take, _partner(p, j, lanes), p) for p in payloads]
    return newkey, newp


def _inclusive_scan(x, lanes):
    n = x.shape[0] * 128
    d = 1
    while d < n:
        if d < 128:
            r = pltpu.roll(x, d, axis=1)
            rprev = jnp.concatenate(
                [jnp.zeros((1, 128), x.dtype), r[:-1]], axis=0)
            sh = jnp.where(lanes < d, rprev, r)
        else:
            m = d // 128
            sh = jnp.concatenate(
                [jnp.zeros((m, 128), x.dtype), x[:-m]], axis=0)
        x = x + sh
        d *= 2
    return x


def _l1_sorted_kernel(cube_ref, part_ref, out_ref):
    pad_k = jnp.full((_PAD_ROWS, 128), _BIG, jnp.float32)
    pad_z = jnp.zeros((_PAD_ROWS, 128), jnp.float32)
    key = jnp.concatenate([cube_ref[_NBLK : 2 * _NBLK, :], pad_k],
                          axis=0)
    u = jnp.concatenate([cube_ref[0:_NBLK, :], pad_z], axis=0)

    rows = lax.broadcasted_iota(jnp.int32, (_RS, 128), 0)
    lanes = lax.broadcasted_iota(jnp.int32, (_RS, 128), 1)
    flat = rows * 128 + lanes

    size = 2
    while size <= _NSORT:
        notasc = (flat & size) != 0
        j = size // 2
        while j >= 1:
            key, (u,) = _substage(key, [u], j, notasc, flat, lanes)
            j //= 2
        size *= 2

    key2 = jnp.concatenate([key, -key], axis=0)
    one = jnp.ones((_RS, 128), jnp.float32)
    f2 = jnp.concatenate([one, 1.0 - one], axis=0)
    u2 = jnp.concatenate([u, u], axis=0)

    rows2 = lax.broadcasted_iota(jnp.int32, (2 * _RS, 128), 0)
    lanes2 = lax.broadcasted_iota(jnp.int32, (2 * _RS, 128), 1)
    flat2 = rows2 * 128 + lanes2

    j = _NSORT
    while j >= 1:
        key2, (f2, u2) = _substage(key2, [f2, u2], j, None, flat2, lanes2)
        j //= 2

    v2 = u2 * jnp.exp(key2 * (1.0 - 2.0 * f2))

    pu = _inclusive_scan(u2 * f2, lanes2)
    pv = _inclusive_scan(v2 * f2, lanes2)
    utot = jnp.sum(u2 * f2, keepdims=True)
    vtot = jnp.sum(v2 * f2, keepdims=True)

    terms = (1.0 - f2) * (u2 * (utot - 2.0 * pu) - v2 * (vtot - 2.0 * pv))
    out_ref[...] = part_ref[...] + 0.00025 * jnp.sum(terms, keepdims=True)


def _run_scalars_cube(ls, lt, tgt):
    return pl.pallas_call(
        _scalars_cube_kernel,
        out_shape=(
            jax.ShapeDtypeStruct((1, 1), jnp.float32),
            jax.ShapeDtypeStruct((2 * _K * _B, _C), jnp.float32),
        ),
        name="ckd_scalars_cube",
    )(ls, lt, tgt)


def _run_l1(cube, part):
    return pl.pallas_call(
        _l1_sorted_kernel,
        out_shape=jax.ShapeDtypeStruct((1, 1), jnp.float32),
        name="ckd_l1_sorted",
    )(cube, part)


def kernel(logits_student, logits_teacher, target):
    ls = logits_student.astype(jnp.float32)
    lt = logits_teacher.astype(jnp.float32)
    tgt = target.astype(jnp.int32).reshape(_B, 1)

    part, cube = _run_scalars_cube(ls, lt, tgt)
    loss = _run_l1(cube.reshape(2 * _NBLK, 128), part)
    return loss.reshape(())


# speedup vs baseline: 65.6578x; 1.0140x over previous
import jax
import jax.numpy as jnp
from jax import lax
from jax.experimental import pallas as pl
from jax.experimental.pallas import tpu as pltpu

_B = 64
_C = 100
_K = 5
_N = _B * _C * _K
_NBLK = _N // 128
_ALPHA = 0.7


def _log_softmax(x):
    m = jnp.max(x, axis=1, keepdims=True)
    s = jnp.exp(x - m)
    return (x - m) - jnp.log(jnp.sum(s, axis=1, keepdims=True))


def _scalars_cube_kernel(ls_ref, lt_ref, tgt_ref, part_ref, cube_ref):
    ls = ls_ref[...]
    lt = lt_ref[...]
    tgt = tgt_ref[...]

    cols = lax.broadcasted_iota(jnp.int32, (_B, _C), 1)
    onehot = cols == tgt
    lp1 = _log_softmax(ls)
    ce = -jnp.sum(jnp.where(onehot, lp1, 0.0), keepdims=True) / float(_B)

    zero = jnp.zeros((1, 1), jnp.float32)
    kd = zero
    tt = zero
    ss = zero
    ts = zero
    sub = zero
    for k in range(1, _K + 1):
        inv = 1.0 / float(k)
        lq = _log_softmax(ls * inv)
        lp = _log_softmax(lt * inv)
        q = jnp.exp(lq)
        p = jnp.exp(lp)
        kl = jnp.sum(p * (lp - lq), keepdims=True) / float(_B * _C)
        kd = kd + kl * (_ALPHA * k * k) + ce * (1.0 - _ALPHA)
        base = (k - 1) * _B
        cube_ref[base : base + _B, :] = p
        cube_ref[_K * _B + base : _K * _B + base + _B, :] = lp - lq
        tt = tt + jnp.sum(p * p, keepdims=True)
        ss = ss + jnp.sum(q * q, keepdims=True)
        ts = ts + jnp.sum(p * q, keepdims=True)
        g = lax.dot_general(p, p, (((1,), (1,)), ((), ())),
                            preferred_element_type=jnp.float32) - \
            lax.dot_general(q, q, (((1,), (1,)), ((), ())),
                            preferred_element_type=jnp.float32)
        h = lax.dot_general(p, p, (((0,), (0,)), ((), ())),
                            preferred_element_type=jnp.float32) - \
            lax.dot_general(q, q, (((0,), (0,)), ((), ())),
                            preferred_element_type=jnp.float32)
        sub = sub + jnp.sum(g * g, keepdims=True) + jnp.sum(h * h, keepdims=True)

    l2 = 0.00025 * (tt * tt - 2.0 * ts * ts + ss * ss)
    part_ref[...] = kd + l2 + sub


_RS = 256
_NSORT = _RS * 128
_PAD_ROWS = _RS - _NBLK
_BIG = 1e30


def _partner(arr, j, lanes):
    if j < 128:
        fwd = pltpu.roll(arr, 128 - j, axis=1)
        bwd = pltpu.roll(arr, j, axis=1)
        return jnp.where((lanes & j) == 0, fwd, bwd)
    s = j // 128
    rr = arr.shape[0]
    y = arr.reshape(rr // (2 * s), 2 * s, 128)
    yswap = jnp.concatenate([y[:, s:, :], y[:, :s, :]], axis=1)
    return yswap.reshape(rr, 128)


def _substage(key, payloads, j, notasc, flat, lanes):
    pk = _partner(key, j, lanes)
    lower = (flat & j) == 0
    conda = lower if notasc is None else jnp.logical_xor(lower, notasc)
    take = (conda & (pk < key)) | (jnp.logical_not(conda) & (pk > key))
    newkey = jnp.where(take, pk, key)
    newp = [jnp.where(take, _partner(p, j, lanes), p) for p in payloads]
    return newkey, newp


def _inclusive_scan(x, lanes):
    n = x.shape[0] * 128
    d = 1
    while d < n:
        if d < 128:
            r = pltpu.roll(x, d, axis=1)
            rprev = jnp.concatenate(
                [jnp.zeros((1, 128), x.dtype), r[:-1]], axis=0)
            sh = jnp.where(lanes < d, rprev, r)
        else:
            m = d // 128
            sh = jnp.concatenate(
                [jnp.zeros((m, 128), x.dtype), x[:-m]], axis=0)
        x = x + sh
        d *= 2
    return x


def _l1_sorted_kernel(cube_ref, part_ref, out_ref):
    pad_k = jnp.full((_PAD_ROWS, 128), _BIG, jnp.float32)
    pad_z = jnp.zeros((_PAD_ROWS, 128), jnp.float32)
    key = jnp.concatenate([cube_ref[_NBLK : 2 * _NBLK, :], pad_k],
                          axis=0)
    u = jnp.concatenate([cube_ref[0:_NBLK, :], pad_z], axis=0)

    rows = lax.broadcasted_iota(jnp.int32, (_RS, 128), 0)
    lanes = lax.broadcasted_iota(jnp.int32, (_RS, 128), 1)
    flat = rows * 128 + lanes

    size = 2
    while size <= _NSORT:
        notasc = None if size == _NSORT else (flat & size) != 0
        j = size // 2
        while j >= 1:
            key, (u,) = _substage(key, [u], j, notasc, flat, lanes)
            j //= 2
        size *= 2

    key2 = jnp.concatenate([key, -key], axis=0)
    one = jnp.ones((_RS, 128), jnp.float32)
    f2 = jnp.concatenate([one, 1.0 - one], axis=0)
    u2 = jnp.concatenate([u, u], axis=0)

    rows2 = lax.broadcasted_iota(jnp.int32, (2 * _RS, 128), 0)
    lanes2 = lax.broadcasted_iota(jnp.int32, (2 * _RS, 128), 1)
    flat2 = rows2 * 128 + lanes2

    j = _NSORT
    while j >= 1:
        key2, (f2, u2) = _substage(key2, [f2, u2], j, None, flat2, lanes2)
        j //= 2

    v2 = u2 * jnp.exp(key2 * (1.0 - 2.0 * f2))

    pu = _inclusive_scan(u2 * f2, lanes2)
    pv = _inclusive_scan(v2 * f2, lanes2)
    utot = jnp.sum(u2 * f2, keepdims=True)
    vtot = jnp.sum(v2 * f2, keepdims=True)

    terms = (1.0 - f2) * (u2 * (utot - 2.0 * pu) - v2 * (vtot - 2.0 * pv))
    out_ref[...] = part_ref[...] + 0.00025 * jnp.sum(terms, keepdims=True)


def _run_scalars_cube(ls, lt, tgt):
    return pl.pallas_call(
        _scalars_cube_kernel,
        out_shape=(
            jax.ShapeDtypeStruct((1, 1), jnp.float32),
            jax.ShapeDtypeStruct((2 * _K * _B, _C), jnp.float32),
        ),
        name="ckd_scalars_cube",
    )(ls, lt, tgt)


def _run_l1(cube, part):
    return pl.pallas_call(
        _l1_sorted_kernel,
        out_shape=jax.ShapeDtypeStruct((1, 1), jnp.float32),
        name="ckd_l1_sorted",
    )(cube, part)


def kernel(logits_student, logits_teacher, target):
    ls = logits_student.astype(jnp.float32)
    lt = logits_teacher.astype(jnp.float32)
    tgt = target.astype(jnp.int32).reshape(_B, 1)

    part, cube = _run_scalars_cube(ls, lt, tgt)
    loss = _run_l1(cube.reshape(2 * _NBLK, 128), part)
    return loss.reshape(())
